# baseline (speedup 1.0000x reference)
import functools

import jax
import jax.numpy as jnp
from jax import lax
from jax.experimental import pallas as pl
from jax.experimental.pallas import tpu as pltpu
from jax.experimental.pallas import tpu_sc as plsc

N_NODES = 10000
N_EDGES = 320000
IN_DIM = 128
OUT_DIM = 16
NUM_HEADS = 8
HD = OUT_DIM * NUM_HEADS

NC, NS = 2, 16
NW = NC * NS
EW = N_EDGES // NW
CHUNK = 400
NCHUNK = EW // CHUNK
CHUNK_B = 200
NCHUNK_B = EW // CHUNK_B
CHUNK_W = 80
NCHUNK_W = EW // CHUNK_W


def _qkv_body(h_ref, w_ref, b_ref, q_ref, k_ref, v_ref):
    acc = jnp.dot(h_ref[...], w_ref[...], preferred_element_type=jnp.float32)
    acc = acc + b_ref[...]
    q_ref[...] = acc[:, :HD]
    k_ref[...] = acc[:, HD:2 * HD]
    v_ref[...] = acc[:, 2 * HD:]


def _qkv(h, w3, b3):
    blk = 1000
    grid = N_NODES // blk
    out = jax.ShapeDtypeStruct((N_NODES, HD), jnp.float32)
    return pl.pallas_call(
        _qkv_body,
        grid=(grid,),
        in_specs=[
            pl.BlockSpec((blk, IN_DIM), lambda i: (i, 0)),
            pl.BlockSpec((IN_DIM, 3 * HD), lambda i: (0, 0)),
            pl.BlockSpec((1, 3 * HD), lambda i: (0, 0)),
        ],
        out_specs=[pl.BlockSpec((blk, HD), lambda i: (i, 0))] * 3,
        out_shape=[out, out, out],
    )(h, w3, b3)


def _kq_body(k_tab, q_tab, src, dst, kq_out,
             ia0, ib0, ia1, ib1, ka0, qa0, ka1, qa1, sa0, sb0, sa1, sb1):
    wid = lax.axis_index("s") * NC + lax.axis_index("c")
    base = wid * EW

    def issue(g, ia, ib, ka, qa, sa, sb):
        off = base + g * CHUNK_B
        pltpu.sync_copy(src.at[pl.ds(off, CHUNK_B)], ia)
        pltpu.sync_copy(dst.at[pl.ds(off, CHUNK_B)], ib)
        pltpu.async_copy(k_tab.at[ia], ka, sa)
        pltpu.async_copy(q_tab.at[ib], qa, sb)

    def drain(ia, ib, ka, qa, sa, sb):
        pltpu.make_async_copy(k_tab.at[ia], ka, sa).wait()
        pltpu.make_async_copy(q_tab.at[ib], qa, sb).wait()

    def compute_dump(g, ka, qa):
        def row(j, c):
            for r in range(HD // 16):
                sl = pl.ds(r * 16, 16)
                ka[j, sl] = ka[j, sl] * qa[j, sl]
            return c

        lax.fori_loop(0, CHUNK_B, row, 0, unroll=2)
        off = base + g * CHUNK_B
        pltpu.sync_copy(ka, kq_out.at[pl.ds(off, CHUNK_B)])

    issue(0, ia0, ib0, ka0, qa0, sa0, sb0)

    def pair(i, carry):
        g0 = i * 2
        issue(g0 + 1, ia1, ib1, ka1, qa1, sa1, sb1)
        drain(ia0, ib0, ka0, qa0, sa0, sb0)
        compute_dump(g0, ka0, qa0)

        @pl.when(g0 + 2 < NCHUNK_B)
        def _():
            issue(g0 + 2, ia0, ib0, ka0, qa0, sa0, sb0)

        drain(ia1, ib1, ka1, qa1, sa1, sb1)
        compute_dump(g0 + 1, ka1, qa1)
        return carry

    lax.fori_loop(0, NCHUNK_B // 2, pair, 0)


def _kq_gather(k_tab, q_tab, src, dst):
    mesh = plsc.VectorSubcoreMesh(core_axis_name="c", subcore_axis_name="s")
    f = functools.partial(
        pl.kernel,
        mesh=mesh,
        out_type=jax.ShapeDtypeStruct((N_EDGES, HD), jnp.float32),
        scratch_types=[
            pltpu.VMEM((CHUNK_B,), jnp.int32),
            pltpu.VMEM((CHUNK_B,), jnp.int32),
            pltpu.VMEM((CHUNK_B,), jnp.int32),
            pltpu.VMEM((CHUNK_B,), jnp.int32),
            pltpu.VMEM((CHUNK_B, HD), jnp.float32),
            pltpu.VMEM((CHUNK_B, HD), jnp.float32),
            pltpu.VMEM((CHUNK_B, HD), jnp.float32),
            pltpu.VMEM((CHUNK_B, HD), jnp.float32),
            pltpu.SemaphoreType.DMA,
            pltpu.SemaphoreType.DMA,
            pltpu.SemaphoreType.DMA,
            pltpu.SemaphoreType.DMA,
        ],
    )(_kq_body)
    return f(k_tab, q_tab, src, dst)


def _edge_body(e_ref, kq_ref, we_ref, be_ref, bd_ref, eo_ref, se_ref):
    pe = jnp.dot(e_ref[...], we_ref[...], preferred_element_type=jnp.float32)
    pe = pe + be_ref[...]
    s = kq_ref[...] * pe * (1.0 / (OUT_DIM ** 0.5))
    eo_ref[...] = s
    hs = jnp.dot(s, bd_ref[...], preferred_element_type=jnp.float32)
    se_ref[...] = jnp.exp(jnp.clip(hs, -5.0, 5.0))


def _edge_stage(e, kq, we, be, bd):
    blk = 1000
    grid = N_EDGES // blk
    return pl.pallas_call(
        _edge_body,
        grid=(grid,),
        in_specs=[
            pl.BlockSpec((blk, IN_DIM), lambda i: (i, 0)),
            pl.BlockSpec((blk, HD), lambda i: (i, 0)),
            pl.BlockSpec((IN_DIM, HD), lambda i: (0, 0)),
            pl.BlockSpec((1, HD), lambda i: (0, 0)),
            pl.BlockSpec((HD, NUM_HEADS), lambda i: (0, 0)),
        ],
        out_specs=[
            pl.BlockSpec((blk, HD), lambda i: (i, 0)),
            pl.BlockSpec((blk, NUM_HEADS), lambda i: (i, 0)),
        ],
        out_shape=[
            jax.ShapeDtypeStruct((N_EDGES, HD), jnp.float32),
            jax.ShapeDtypeStruct((N_EDGES, NUM_HEADS), jnp.float32),
        ],
    )(e, kq, we, be, bd)


def _scat_w_body(v_tab, src, dst, sef, zw_hbm, part_w,
                 ia0, ib0, sf0, va0, ia1, ib1, sf1, va1, acc_w, s0, s1):
    cid = lax.axis_index("c")
    sid = lax.axis_index("s")
    wid = sid * NC + cid
    base = wid * EW

    @pl.when(sid == 0)
    def _():
        pltpu.sync_copy(zw_hbm, acc_w)

    plsc.subcore_barrier()

    def issue(g, ia, ib, sf, va, sem):
        off = base + g * CHUNK_W
        pltpu.sync_copy(src.at[pl.ds(off, CHUNK_W)], ia)
        pltpu.sync_copy(dst.at[pl.ds(off, CHUNK_W)], ib)
        pltpu.sync_copy(sef.at[pl.ds(off * NUM_HEADS, CHUNK_W * NUM_HEADS)], sf)
        pltpu.async_copy(v_tab.at[ia], va, sem)

    def scale_scatter(sf, va, ib):
        def pair(jp, c):
            j0 = jp * 2
            se2 = sf[pl.ds(jp * 16, 16)]
            for parity in range(2):
                j = j0 + parity
                lb = parity * NUM_HEADS
                for h in range(NUM_HEADS):
                    sev = jnp.full((16,), se2[lb + h], dtype=jnp.float32)
                    sl = pl.ds(h * 16, 16)
                    va[j, sl] = va[j, sl] * sev
            return c

        lax.fori_loop(0, CHUNK_W // 2, pair, 0)
        pltpu.sync_copy(va, acc_w.at[ib], add=True)

    issue(0, ia0, ib0, sf0, va0, s0)

    def pairloop(i, carry):
        g0 = i * 2
        issue(g0 + 1, ia1, ib1, sf1, va1, s1)
        pltpu.make_async_copy(v_tab.at[ia0], va0, s0).wait()
        scale_scatter(sf0, va0, ib0)

        @pl.when(g0 + 2 < NCHUNK_W)
        def _():
            issue(g0 + 2, ia0, ib0, sf0, va0, s0)

        pltpu.make_async_copy(v_tab.at[ia1], va1, s1).wait()
        scale_scatter(sf1, va1, ib1)
        return carry

    lax.fori_loop(0, NCHUNK_W // 2, pairloop, 0)
    pltpu.make_async_copy(v_tab.at[ia0], va0, s0).wait()
    scale_scatter(sf0, va0, ib0)
    plsc.subcore_barrier()

    per = 624
    @pl.when(sid < NS - 1)
    def _():
        pltpu.sync_copy(acc_w.at[pl.ds(sid * per, per)],
                        part_w.at[cid, pl.ds(sid * per, per)])

    @pl.when(sid == NS - 1)
    def _():
        rest = N_NODES - per * (NS - 1)
        pltpu.sync_copy(acc_w.at[pl.ds((NS - 1) * per, rest)],
                        part_w.at[cid, pl.ds((NS - 1) * per, rest)])


def _scat_w_stage(v_tab, src, dst, sef, zw):
    mesh = plsc.VectorSubcoreMesh(core_axis_name="c", subcore_axis_name="s")
    f = functools.partial(
        pl.kernel,
        mesh=mesh,
        out_type=jax.ShapeDtypeStruct((NC, N_NODES, HD), jnp.float32),
        scratch_types=[
            pltpu.VMEM((CHUNK_W,), jnp.int32),
            pltpu.VMEM((CHUNK_W,), jnp.int32),
            pltpu.VMEM((CHUNK_W * NUM_HEADS,), jnp.float32),
            pltpu.VMEM((CHUNK_W, HD), jnp.float32),
            pltpu.VMEM((CHUNK_W,), jnp.int32),
            pltpu.VMEM((CHUNK_W,), jnp.int32),
            pltpu.VMEM((CHUNK_W * NUM_HEADS,), jnp.float32),
            pltpu.VMEM((CHUNK_W, HD), jnp.float32),
            pltpu.VMEM_SHARED((N_NODES, HD), jnp.float32),
            pltpu.SemaphoreType.DMA,
            pltpu.SemaphoreType.DMA,
        ],
    )(_scat_w_body)
    return f(v_tab, src, dst, sef, zw)


def _scat_z_body(dst, sef, zz_hbm, zm_hbm, part_z, idx_b, sef_buf, acc_z, zm_buf):
    cid = lax.axis_index("c")
    sid = lax.axis_index("s")
    wid = sid * NC + cid
    base = wid * EW

    pltpu.sync_copy(zz_hbm, acc_z)
    pltpu.sync_copy(zm_hbm, zm_buf)

    def chunk(g, carry):
        off = base + g * CHUNK
        pltpu.sync_copy(dst.at[pl.ds(off, CHUNK)], idx_b)
        pltpu.sync_copy(sef.at[pl.ds(off * NUM_HEADS, CHUNK * NUM_HEADS)],
                        sef_buf.at[pl.ds(0, CHUNK * NUM_HEADS)])

        def grp(jp, c):
            dv8 = idx_b[pl.ds(jp * 16, 16)] * NUM_HEADS
            zmask = zm_buf[...]
            for l in range(16):
                j16 = jp * 16 + l
                sv = sef_buf[pl.ds(j16 * NUM_HEADS, 16)] * zmask
                o = dv8[l]
                acc_z[pl.ds(o, 16)] = acc_z[pl.ds(o, 16)] + sv
            return c

        lax.fori_loop(0, CHUNK // 16, grp, 0)
        return carry

    lax.fori_loop(0, NCHUNK, chunk, 0)
    pltpu.sync_copy(acc_z.at[pl.ds(0, N_NODES * NUM_HEADS)], part_z.at[wid])


def _scat_z_stage(dst, sef, zz, zm):
    mesh = plsc.VectorSubcoreMesh(core_axis_name="c", subcore_axis_name="s")
    f = functools.partial(
        pl.kernel,
        mesh=mesh,
        out_type=jax.ShapeDtypeStruct((NW, N_NODES * NUM_HEADS), jnp.float32),
        scratch_types=[
            pltpu.VMEM((CHUNK,), jnp.int32),
            pltpu.VMEM((CHUNK * NUM_HEADS + 16,), jnp.float32),
            pltpu.VMEM((N_NODES * NUM_HEADS + 16,), jnp.float32),
            pltpu.VMEM((16,), jnp.float32),
        ],
    )(_scat_z_body)
    return f(dst, sef, zz, zm)


def _comb_body(w_ref, z_ref, bd8_ref, out_ref):
    w = w_ref[0] + w_ref[1]
    z = jnp.dot(jnp.sum(z_ref[...], axis=0), bd8_ref[...],
                preferred_element_type=jnp.float32)
    out_ref[...] = w / (z + 1e-6)


def _combine(w2, z2, bd8):
    blk = 1000
    grid = N_NODES // blk
    return pl.pallas_call(
        _comb_body,
        grid=(grid,),
        in_specs=[
            pl.BlockSpec((NC, blk, HD), lambda i: (0, i, 0)),
            pl.BlockSpec((NW, blk, NUM_HEADS), lambda i: (0, i, 0)),
            pl.BlockSpec((NUM_HEADS, HD), lambda i: (0, 0)),
        ],
        out_specs=pl.BlockSpec((blk, HD), lambda i: (i, 0)),
        out_shape=jax.ShapeDtypeStruct((N_NODES, HD), jnp.float32),
    )(w2, z2, bd8)


def kernel(h, e, edge_index, Wq, bq, Wk, bk, Wv, bv, We, be):
    w3 = jnp.concatenate([Wq, Wk, Wv], axis=1)
    b3 = jnp.concatenate([bq, bk, bv]).reshape(1, 3 * HD)
    src = edge_index[0]
    dst = edge_index[1]
    bd = jnp.repeat(jnp.eye(NUM_HEADS, dtype=jnp.float32), OUT_DIM, axis=0)
    bd8 = bd.T
    zw = jnp.zeros((N_NODES, HD), jnp.float32)
    zz = jnp.zeros((N_NODES * NUM_HEADS + 16,), jnp.float32)

    q_tab, k_tab, v_tab = _qkv(h, w3, b3)
    kq = _kq_gather(k_tab, q_tab, src, dst)
    e_out, se = _edge_stage(e, kq, We, be.reshape(1, HD), bd)
    part_w = _scat_w_stage(v_tab, src, dst, se.reshape(-1), zw)
    zm = (jnp.arange(16) < NUM_HEADS).astype(jnp.float32)
    part_z = _scat_z_stage(dst, se.reshape(-1), zz, zm)
    h_out = _combine(part_w,
                     part_z.reshape(NW, N_NODES, NUM_HEADS), bd8)

    return (h_out.reshape(N_NODES, NUM_HEADS, OUT_DIM),
            e_out.reshape(N_EDGES, NUM_HEADS, OUT_DIM))


# speedup vs baseline: 1.0141x; 1.0141x over previous
import functools

import jax
import jax.numpy as jnp
from jax import lax
from jax.experimental import pallas as pl
from jax.experimental.pallas import tpu as pltpu
from jax.experimental.pallas import tpu_sc as plsc

N_NODES = 10000
N_EDGES = 320000
IN_DIM = 128
OUT_DIM = 16
NUM_HEADS = 8
HD = OUT_DIM * NUM_HEADS

NC, NS = 2, 16
NW = NC * NS
EW = N_EDGES // NW
CHUNK = 400
NCHUNK = EW // CHUNK
CHUNK_B = 200
NCHUNK_B = EW // CHUNK_B
CHUNK_W = 200
NCHUNK_W = EW // CHUNK_W


def _qkv_body(h_ref, w_ref, b_ref, q_ref, k_ref, v_ref):
    acc = jnp.dot(h_ref[...], w_ref[...], preferred_element_type=jnp.float32)
    acc = acc + b_ref[...]
    q_ref[...] = acc[:, :HD]
    k_ref[...] = acc[:, HD:2 * HD]
    v_ref[...] = acc[:, 2 * HD:]


def _qkv(h, w3, b3):
    blk = 1000
    grid = N_NODES // blk
    out = jax.ShapeDtypeStruct((N_NODES, HD), jnp.float32)
    return pl.pallas_call(
        _qkv_body,
        grid=(grid,),
        in_specs=[
            pl.BlockSpec((blk, IN_DIM), lambda i: (i, 0)),
            pl.BlockSpec((IN_DIM, 3 * HD), lambda i: (0, 0)),
            pl.BlockSpec((1, 3 * HD), lambda i: (0, 0)),
        ],
        out_specs=[pl.BlockSpec((blk, HD), lambda i: (i, 0))] * 3,
        out_shape=[out, out, out],
    )(h, w3, b3)


def _kq_body(k_tab, q_tab, src, dst, kq_out,
             ia0, ib0, ia1, ib1, ka0, qa0, ka1, qa1, sa0, sb0, sa1, sb1):
    wid = lax.axis_index("s") * NC + lax.axis_index("c")
    base = wid * EW

    def issue(g, ia, ib, ka, qa, sa, sb):
        off = base + g * CHUNK_B
        pltpu.sync_copy(src.at[pl.ds(off, CHUNK_B)], ia)
        pltpu.sync_copy(dst.at[pl.ds(off, CHUNK_B)], ib)
        pltpu.async_copy(k_tab.at[ia], ka, sa)
        pltpu.async_copy(q_tab.at[ib], qa, sb)

    def drain(ia, ib, ka, qa, sa, sb):
        pltpu.make_async_copy(k_tab.at[ia], ka, sa).wait()
        pltpu.make_async_copy(q_tab.at[ib], qa, sb).wait()

    def compute_dump(g, ka, qa):
        def row(j, c):
            for r in range(HD // 16):
                sl = pl.ds(r * 16, 16)
                ka[j, sl] = ka[j, sl] * qa[j, sl]
            return c

        lax.fori_loop(0, CHUNK_B, row, 0, unroll=2)
        off = base + g * CHUNK_B
        pltpu.sync_copy(ka, kq_out.at[pl.ds(off, CHUNK_B)])

    issue(0, ia0, ib0, ka0, qa0, sa0, sb0)

    def pair(i, carry):
        g0 = i * 2
        issue(g0 + 1, ia1, ib1, ka1, qa1, sa1, sb1)
        drain(ia0, ib0, ka0, qa0, sa0, sb0)
        compute_dump(g0, ka0, qa0)

        @pl.when(g0 + 2 < NCHUNK_B)
        def _():
            issue(g0 + 2, ia0, ib0, ka0, qa0, sa0, sb0)

        drain(ia1, ib1, ka1, qa1, sa1, sb1)
        compute_dump(g0 + 1, ka1, qa1)
        return carry

    lax.fori_loop(0, NCHUNK_B // 2, pair, 0)


def _kq_gather(k_tab, q_tab, src, dst):
    mesh = plsc.VectorSubcoreMesh(core_axis_name="c", subcore_axis_name="s")
    f = functools.partial(
        pl.kernel,
        mesh=mesh,
        out_type=jax.ShapeDtypeStruct((N_EDGES, HD), jnp.float32),
        scratch_types=[
            pltpu.VMEM((CHUNK_B,), jnp.int32),
            pltpu.VMEM((CHUNK_B,), jnp.int32),
            pltpu.VMEM((CHUNK_B,), jnp.int32),
            pltpu.VMEM((CHUNK_B,), jnp.int32),
            pltpu.VMEM((CHUNK_B, HD), jnp.float32),
            pltpu.VMEM((CHUNK_B, HD), jnp.float32),
            pltpu.VMEM((CHUNK_B, HD), jnp.float32),
            pltpu.VMEM((CHUNK_B, HD), jnp.float32),
            pltpu.SemaphoreType.DMA,
            pltpu.SemaphoreType.DMA,
            pltpu.SemaphoreType.DMA,
            pltpu.SemaphoreType.DMA,
        ],
    )(_kq_body)
    return f(k_tab, q_tab, src, dst)


def _edge_body(e_ref, kq_ref, we_ref, be_ref, bd_ref, eo_ref, se_ref):
    pe = jnp.dot(e_ref[...], we_ref[...], preferred_element_type=jnp.float32)
    pe = pe + be_ref[...]
    s = kq_ref[...] * pe * (1.0 / (OUT_DIM ** 0.5))
    eo_ref[...] = s
    hs = jnp.dot(s, bd_ref[...], preferred_element_type=jnp.float32)
    se_ref[...] = jnp.exp(jnp.clip(hs, -5.0, 5.0))


def _edge_stage(e, kq, we, be, bd):
    blk = 1000
    grid = N_EDGES // blk
    return pl.pallas_call(
        _edge_body,
        grid=(grid,),
        in_specs=[
            pl.BlockSpec((blk, IN_DIM), lambda i: (i, 0)),
            pl.BlockSpec((blk, HD), lambda i: (i, 0)),
            pl.BlockSpec((IN_DIM, HD), lambda i: (0, 0)),
            pl.BlockSpec((1, HD), lambda i: (0, 0)),
            pl.BlockSpec((HD, NUM_HEADS), lambda i: (0, 0)),
        ],
        out_specs=[
            pl.BlockSpec((blk, HD), lambda i: (i, 0)),
            pl.BlockSpec((blk, NUM_HEADS), lambda i: (i, 0)),
        ],
        out_shape=[
            jax.ShapeDtypeStruct((N_EDGES, HD), jnp.float32),
            jax.ShapeDtypeStruct((N_EDGES, NUM_HEADS), jnp.float32),
        ],
    )(e, kq, we, be, bd)


def _scat_w_body(v_tab, src, dst, sef, zw_hbm, part_w,
                 idx_a, idx_b, sef_buf, va, acc_w, sem_a):
    cid = lax.axis_index("c")
    sid = lax.axis_index("s")
    wid = sid * NC + cid
    base = wid * EW

    @pl.when(sid == 0)
    def _():
        pltpu.sync_copy(zw_hbm, acc_w)

    plsc.subcore_barrier()

    def chunk(g, carry):
        off = base + g * CHUNK_W
        pltpu.sync_copy(src.at[pl.ds(off, CHUNK_W)], idx_a)
        pltpu.sync_copy(dst.at[pl.ds(off, CHUNK_W)], idx_b)
        pltpu.sync_copy(sef.at[pl.ds(off * NUM_HEADS, CHUNK_W * NUM_HEADS)],
                        sef_buf)
        pltpu.async_copy(v_tab.at[idx_a], va, sem_a).wait()

        def pair(jp, c):
            j0 = jp * 2
            se2 = sef_buf[pl.ds(jp * 16, 16)]
            for parity in range(2):
                j = j0 + parity
                lb = parity * NUM_HEADS
                for h in range(NUM_HEADS):
                    sev = jnp.full((16,), se2[lb + h], dtype=jnp.float32)
                    sl = pl.ds(h * 16, 16)
                    va[j, sl] = va[j, sl] * sev
            return c

        lax.fori_loop(0, CHUNK_W // 2, pair, 0)
        pltpu.sync_copy(va, acc_w.at[idx_b], add=True)
        return carry

    lax.fori_loop(0, NCHUNK_W, chunk, 0)
    plsc.subcore_barrier()

    per = 624
    @pl.when(sid < NS - 1)
    def _():
        pltpu.sync_copy(acc_w.at[pl.ds(sid * per, per)],
                        part_w.at[cid, pl.ds(sid * per, per)])

    @pl.when(sid == NS - 1)
    def _():
        rest = N_NODES - per * (NS - 1)
        pltpu.sync_copy(acc_w.at[pl.ds((NS - 1) * per, rest)],
                        part_w.at[cid, pl.ds((NS - 1) * per, rest)])


def _scat_w_stage(v_tab, src, dst, sef, zw):
    mesh = plsc.VectorSubcoreMesh(core_axis_name="c", subcore_axis_name="s")
    f = functools.partial(
        pl.kernel,
        mesh=mesh,
        out_type=jax.ShapeDtypeStruct((NC, N_NODES, HD), jnp.float32),
        scratch_types=[
            pltpu.VMEM((CHUNK_W,), jnp.int32),
            pltpu.VMEM((CHUNK_W,), jnp.int32),
            pltpu.VMEM((CHUNK_W * NUM_HEADS,), jnp.float32),
            pltpu.VMEM((CHUNK_W, HD), jnp.float32),
            pltpu.VMEM_SHARED((N_NODES, HD), jnp.float32),
            pltpu.SemaphoreType.DMA,
        ],
    )(_scat_w_body)
    return f(v_tab, src, dst, sef, zw)


def _scat_z_body(dst, sef, zz_hbm, zm_hbm, part_z, idx_b, sef_buf, acc_z, zm_buf):
    cid = lax.axis_index("c")
    sid = lax.axis_index("s")
    wid = sid * NC + cid
    base = wid * EW

    pltpu.sync_copy(zz_hbm, acc_z)
    pltpu.sync_copy(zm_hbm, zm_buf)

    def chunk(g, carry):
        off = base + g * CHUNK
        pltpu.sync_copy(dst.at[pl.ds(off, CHUNK)], idx_b)
        pltpu.sync_copy(sef.at[pl.ds(off * NUM_HEADS, CHUNK * NUM_HEADS)],
                        sef_buf.at[pl.ds(0, CHUNK * NUM_HEADS)])

        def grp(jp, c):
            dv8 = idx_b[pl.ds(jp * 16, 16)] * NUM_HEADS
            zmask = zm_buf[...]
            for l in range(16):
                j16 = jp * 16 + l
                sv = sef_buf[pl.ds(j16 * NUM_HEADS, 16)] * zmask
                o = dv8[l]
                acc_z[pl.ds(o, 16)] = acc_z[pl.ds(o, 16)] + sv
            return c

        lax.fori_loop(0, CHUNK // 16, grp, 0)
        return carry

    lax.fori_loop(0, NCHUNK, chunk, 0)
    pltpu.sync_copy(acc_z.at[pl.ds(0, N_NODES * NUM_HEADS)], part_z.at[wid])


def _scat_z_stage(dst, sef, zz, zm):
    mesh = plsc.VectorSubcoreMesh(core_axis_name="c", subcore_axis_name="s")
    f = functools.partial(
        pl.kernel,
        mesh=mesh,
        out_type=jax.ShapeDtypeStruct((NW, N_NODES * NUM_HEADS), jnp.float32),
        scratch_types=[
            pltpu.VMEM((CHUNK,), jnp.int32),
            pltpu.VMEM((CHUNK * NUM_HEADS + 16,), jnp.float32),
            pltpu.VMEM((N_NODES * NUM_HEADS + 16,), jnp.float32),
            pltpu.VMEM((16,), jnp.float32),
        ],
    )(_scat_z_body)
    return f(dst, sef, zz, zm)


def _comb_body(w_ref, z_ref, bd8_ref, out_ref):
    w = w_ref[0] + w_ref[1]
    z = jnp.dot(jnp.sum(z_ref[...], axis=0), bd8_ref[...],
                preferred_element_type=jnp.float32)
    out_ref[...] = w / (z + 1e-6)


def _combine(w2, z2, bd8):
    blk = 1000
    grid = N_NODES // blk
    return pl.pallas_call(
        _comb_body,
        grid=(grid,),
        in_specs=[
            pl.BlockSpec((NC, blk, HD), lambda i: (0, i, 0)),
            pl.BlockSpec((NW, blk, NUM_HEADS), lambda i: (0, i, 0)),
            pl.BlockSpec((NUM_HEADS, HD), lambda i: (0, 0)),
        ],
        out_specs=pl.BlockSpec((blk, HD), lambda i: (i, 0)),
        out_shape=jax.ShapeDtypeStruct((N_NODES, HD), jnp.float32),
    )(w2, z2, bd8)


def kernel(h, e, edge_index, Wq, bq, Wk, bk, Wv, bv, We, be):
    w3 = jnp.concatenate([Wq, Wk, Wv], axis=1)
    b3 = jnp.concatenate([bq, bk, bv]).reshape(1, 3 * HD)
    src = edge_index[0]
    dst = edge_index[1]
    bd = jnp.repeat(jnp.eye(NUM_HEADS, dtype=jnp.float32), OUT_DIM, axis=0)
    bd8 = bd.T
    zw = jnp.zeros((N_NODES, HD), jnp.float32)
    zz = jnp.zeros((N_NODES * NUM_HEADS + 16,), jnp.float32)

    q_tab, k_tab, v_tab = _qkv(h, w3, b3)
    kq = _kq_gather(k_tab, q_tab, src, dst)
    e_out, se = _edge_stage(e, kq, We, be.reshape(1, HD), bd)
    part_w = _scat_w_stage(v_tab, src, dst, se.reshape(-1), zw)
    zm = (jnp.arange(16) < NUM_HEADS).astype(jnp.float32)
    part_z = _scat_z_stage(dst, se.reshape(-1), zz, zm)
    h_out = _combine(part_w,
                     part_z.reshape(NW, N_NODES, NUM_HEADS), bd8)

    return (h_out.reshape(N_NODES, NUM_HEADS, OUT_DIM),
            e_out.reshape(N_EDGES, NUM_HEADS, OUT_DIM))
